# Initial kernel scaffold; baseline (speedup 1.0000x reference)
import jax
import jax.numpy as jnp
from jax.experimental import pallas as pl


def kernel(query_embedding, memory_embeddings, memory_som_masks, W, b, k):
    raise NotImplementedError("write your pallas kernel here")


# speedup vs baseline: 8.7809x; 8.7809x over previous
import functools

import jax
import jax.numpy as jnp
from jax import lax
from jax.experimental import pallas as pl
from jax.experimental.pallas import tpu as pltpu
from jax.experimental.pallas import tpu_sc as plsc

_K = 8


def _topk_body(q_ref, mem_ref, w_ref, b_ref, wout_ref, iout_ref, mh_ref, mn_ref):
    f32 = jnp.float32
    m_rows = mem_ref.shape[0]

    @pl.when(pl.program_id(0) == 0)
    def _():
        t = jnp.tanh(
            lax.dot_general(mem_ref[...], w_ref[...], (((1,), (1,)), ((), ())),
                            preferred_element_type=f32)
            + b_ref[...])
        nsq = jnp.sum(t * t, axis=1, keepdims=True)
        norm = jnp.sqrt(nsq)
        scale = jnp.where(norm > 0.95, 0.95 / norm, jnp.ones_like(norm))
        mh = t * scale
        mh_ref[...] = mh
        ones = jnp.ones((1, mh.shape[1]), f32)
        mn_ref[...] = lax.dot_general(
            ones, mh * mh, (((1,), (1,)), ((), ())),
            precision=lax.Precision.HIGHEST, preferred_element_type=f32)

    t = jnp.tanh(
        lax.dot_general(q_ref[...], w_ref[...], (((1,), (1,)), ((), ())),
                        preferred_element_type=f32)
        + b_ref[...])
    qnsq = jnp.sum(t * t, axis=1, keepdims=True)
    qnorm = jnp.sqrt(qnsq)
    qscale = jnp.where(qnorm > 0.95, 0.95 / qnorm, jnp.ones_like(qnorm))
    qh = t * qscale
    qn = qnsq * qscale * qscale

    gram = lax.dot_general(qh, mh_ref[...], (((1,), (1,)), ((), ())),
                           precision=lax.Precision.HIGHEST,
                           preferred_element_type=f32)
    mn = mn_ref[...]
    diff_nsq = qn + mn - 2.0 * gram
    denom = (1.0 - qn) * (1.0 - mn)
    arg = 1.0 + 2.0 * diff_nsq / (denom + 1e-08)
    arg = jnp.maximum(arg, 1.0 + 1e-06)

    iota = lax.broadcasted_iota(jnp.int32, arg.shape, 1)
    vals, idxs = [], []
    a = arg
    for _ in range(_K):
        mval = jnp.min(a, axis=1, keepdims=True)
        cand = jnp.where(a == mval, iota, jnp.int32(m_rows))
        ix = jnp.min(cand, axis=1, keepdims=True)
        vals.append(mval)
        idxs.append(ix)
        a = jnp.where(iota == ix, jnp.float32(jnp.inf), a)
    v = jnp.concatenate(vals, axis=1)
    ii = jnp.concatenate(idxs, axis=1)

    dist = jnp.log(v + jnp.sqrt((v - 1.0) * (v + 1.0)))
    mmax = jnp.max(-dist, axis=1, keepdims=True)
    e = jnp.exp(-dist - mmax)
    wout_ref[...] = e / jnp.sum(e, axis=1, keepdims=True)
    iout_ref[...] = ii


def _topk_weights(q, mem, w, b_row, qblk):
    bq, d = q.shape
    m = mem.shape[0]
    return pl.pallas_call(
        _topk_body,
        grid=(bq // qblk,),
        in_specs=[
            pl.BlockSpec((qblk, d), lambda i: (i, 0)),
            pl.BlockSpec((m, d), lambda i: (0, 0)),
            pl.BlockSpec((d, d), lambda i: (0, 0)),
            pl.BlockSpec((1, d), lambda i: (0, 0)),
        ],
        out_specs=[
            pl.BlockSpec((qblk, _K), lambda i: (i, 0)),
            pl.BlockSpec((qblk, _K), lambda i: (i, 0)),
        ],
        out_shape=[
            jax.ShapeDtypeStruct((bq, _K), jnp.float32),
            jax.ShapeDtypeStruct((bq, _K), jnp.int32),
        ],
        scratch_shapes=[
            pltpu.VMEM((m, d), jnp.float32),
            pltpu.VMEM((1, m), jnp.float32),
        ],
    )(q, mem, w, b_row)


def _sc_gather(table, idx):
    rows_total, width = idx.shape[0], table.shape[1]
    info = plsc.get_sparse_core_info()
    ncores, nsub = info.num_cores, info.num_subcores
    nworkers = ncores * nsub
    bpw = rows_total // nworkers
    mesh = plsc.VectorSubcoreMesh(core_axis_name="c", subcore_axis_name="s")

    @functools.partial(
        pl.kernel, mesh=mesh,
        out_type=jax.ShapeDtypeStruct((rows_total, width), jnp.float32),
        scratch_types=[
            pltpu.VMEM((bpw,), jnp.int32),
            pltpu.VMEM((bpw, width), jnp.float32),
            pltpu.SemaphoreType.DMA,
        ],
    )
    def gather_kernel(table_hbm, idx_hbm, out_hbm, idx_v, rows_v, sem):
        wid = lax.axis_index("s") * ncores + lax.axis_index("c")
        base = wid * bpw
        pltpu.sync_copy(idx_hbm.at[pl.ds(base, bpw)], idx_v)
        pltpu.async_copy(table_hbm.at[idx_v], rows_v, sem).wait()
        pltpu.sync_copy(rows_v, out_hbm.at[pl.ds(base, bpw)])

    return gather_kernel(table, idx)


def kernel(query_embedding, memory_embeddings, memory_som_masks, W, b, k):
    bq = query_embedding.shape[0]
    mask_w = memory_som_masks.shape[1]
    weights, top_idx = _topk_weights(
        query_embedding, memory_embeddings, W, b.reshape(1, -1), qblk=128)
    pad_w = (-mask_w) % 128
    masks_padded = (jnp.pad(memory_som_masks, ((0, 0), (0, pad_w)))
                    if pad_w else memory_som_masks)
    rows = _sc_gather(masks_padded, top_idx.reshape(-1))
    return weights, rows[:, :mask_w].reshape(bq, _K, mask_w)


# speedup vs baseline: 8.8329x; 1.0059x over previous
import functools

import jax
import jax.numpy as jnp
from jax import lax
from jax.experimental import pallas as pl
from jax.experimental.pallas import tpu as pltpu
from jax.experimental.pallas import tpu_sc as plsc

_K = 8


def _topk_body(q_ref, mem_ref, w_ref, b_ref, wout_ref, iout_ref, mh_ref, mn_ref):
    f32 = jnp.float32
    m_rows = mem_ref.shape[0]

    @pl.when(pl.program_id(0) == 0)
    def _():
        t = jnp.tanh(
            lax.dot_general(mem_ref[...], w_ref[...], (((1,), (1,)), ((), ())),
                            preferred_element_type=f32)
            + b_ref[...])
        nsq = jnp.sum(t * t, axis=1, keepdims=True)
        norm = jnp.sqrt(nsq)
        scale = jnp.where(norm > 0.95, 0.95 / norm, jnp.ones_like(norm))
        mh = t * scale
        mh_ref[...] = mh
        ones = jnp.ones((1, mh.shape[1]), f32)
        mn_ref[...] = lax.dot_general(
            ones, mh * mh, (((1,), (1,)), ((), ())),
            precision=lax.Precision.HIGHEST, preferred_element_type=f32)

    t = jnp.tanh(
        lax.dot_general(q_ref[...], w_ref[...], (((1,), (1,)), ((), ())),
                        preferred_element_type=f32)
        + b_ref[...])
    qnsq = jnp.sum(t * t, axis=1, keepdims=True)
    qnorm = jnp.sqrt(qnsq)
    qscale = jnp.where(qnorm > 0.95, 0.95 / qnorm, jnp.ones_like(qnorm))
    qh = t * qscale
    qn = qnsq * qscale * qscale

    gram = lax.dot_general(qh, mh_ref[...], (((1,), (1,)), ((), ())),
                           precision=lax.Precision.HIGHEST,
                           preferred_element_type=f32)
    mn = mn_ref[...]
    diff_nsq = qn + mn - 2.0 * gram
    denom = (1.0 - qn) * (1.0 - mn)
    arg = 1.0 + 2.0 * diff_nsq / (denom + 1e-08)
    arg = jnp.maximum(arg, 1.0 + 1e-06)

    iota = lax.broadcasted_iota(jnp.int32, arg.shape, 1)
    vals, idxs = [], []
    a = arg
    for _ in range(_K):
        mval = jnp.min(a, axis=1, keepdims=True)
        cand = jnp.where(a == mval, iota, jnp.int32(m_rows))
        ix = jnp.min(cand, axis=1, keepdims=True)
        vals.append(mval)
        idxs.append(ix)
        a = jnp.where(iota == ix, jnp.float32(jnp.inf), a)
    v = jnp.concatenate(vals, axis=1)
    ii = jnp.concatenate(idxs, axis=1)

    dist = jnp.log(v + jnp.sqrt((v - 1.0) * (v + 1.0)))
    mmax = jnp.max(-dist, axis=1, keepdims=True)
    e = jnp.exp(-dist - mmax)
    wout_ref[...] = e / jnp.sum(e, axis=1, keepdims=True)
    iout_ref[...] = ii


def _topk_weights(q, mem, w, b_row, qblk):
    bq, d = q.shape
    m = mem.shape[0]
    return pl.pallas_call(
        _topk_body,
        grid=(bq // qblk,),
        in_specs=[
            pl.BlockSpec((qblk, d), lambda i: (i, 0)),
            pl.BlockSpec((m, d), lambda i: (0, 0)),
            pl.BlockSpec((d, d), lambda i: (0, 0)),
            pl.BlockSpec((1, d), lambda i: (0, 0)),
        ],
        out_specs=[
            pl.BlockSpec((qblk, _K), lambda i: (i, 0)),
            pl.BlockSpec((qblk, _K), lambda i: (i, 0)),
        ],
        out_shape=[
            jax.ShapeDtypeStruct((bq, _K), jnp.float32),
            jax.ShapeDtypeStruct((bq, _K), jnp.int32),
        ],
        scratch_shapes=[
            pltpu.VMEM((m, d), jnp.float32),
            pltpu.VMEM((1, m), jnp.float32),
        ],
    )(q, mem, w, b_row)


def _sc_gather(table, idx):
    rows_total, width = idx.shape[0], table.shape[1]
    nrows = table.shape[0]
    info = plsc.get_sparse_core_info()
    ncores, nsub = info.num_cores, info.num_subcores
    nworkers = ncores * nsub
    bpw = rows_total // nworkers
    rows_per_sub = nrows // nsub
    mesh = plsc.VectorSubcoreMesh(core_axis_name="c", subcore_axis_name="s")

    @functools.partial(
        pl.kernel, mesh=mesh,
        out_type=jax.ShapeDtypeStruct((rows_total, width), jnp.float32),
        scratch_types=[
            pltpu.VMEM((bpw,), jnp.int32),
            pltpu.VMEM((bpw, width), jnp.float32),
            pltpu.SemaphoreType.DMA,
        ],
    )
    def gather_kernel(table_hbm, idx_hbm, out_hbm, idx_v, rows_v, sem):
        sid = lax.axis_index("s")
        wid = sid * ncores + lax.axis_index("c")
        base = wid * bpw
        pltpu.sync_copy(idx_hbm.at[pl.ds(base, bpw)], idx_v)

        def fire_group(g, carry):
            vec = idx_v[pl.ds(g * 16, 16)]
            for jj in range(16):
                pltpu.make_async_copy(
                    table_hbm.at[vec[jj]], rows_v.at[g * 16 + jj], sem).start()
            return carry

        lax.fori_loop(0, bpw // 16, fire_group, 0)

        def drain(i, carry):
            pltpu.make_async_copy(table_hbm.at[0], rows_v.at[i], sem).wait()
            return carry

        lax.fori_loop(0, bpw, drain, 0)
        pltpu.sync_copy(rows_v, out_hbm.at[pl.ds(base, bpw)])

    return gather_kernel(table, idx)


def kernel(query_embedding, memory_embeddings, memory_som_masks, W, b, k):
    bq = query_embedding.shape[0]
    mask_w = memory_som_masks.shape[1]
    weights, top_idx = _topk_weights(
        query_embedding, memory_embeddings, W, b.reshape(1, -1), qblk=128)
    rows = _sc_gather(memory_som_masks, top_idx.reshape(-1))
    return weights, rows.reshape(bq, _K, mask_w)
